# Initial kernel scaffold; baseline (speedup 1.0000x reference)
import jax
import jax.numpy as jnp
from jax.experimental import pallas as pl


def kernel(model, prompt, category, P, Q, W_text, b_text, cat_emb, W_cls, b_cls):
    raise NotImplementedError("write your pallas kernel here")


# speedup vs baseline: 4.1570x; 4.1570x over previous
import functools

import jax
import jax.numpy as jnp
from jax import lax
from jax.experimental import pallas as pl
from jax.experimental.pallas import tpu as pltpu
from jax.experimental.pallas import tpu_sc as plsc

B = 16384
DIM = 64
TEXT_DIM = 768

NC = 2
NS = 16
NW = NC * NS
BPW = B // NW
CH = 64


def _sc_gather(Q, prompt, P, model, cat_emb, category):
    mesh = plsc.VectorSubcoreMesh(core_axis_name="c", subcore_axis_name="s")

    @functools.partial(
        pl.kernel,
        mesh=mesh,
        out_type=[
            jax.ShapeDtypeStruct((B, TEXT_DIM), jnp.float32),
            jax.ShapeDtypeStruct((B, 2 * DIM), jnp.float32),
            jax.ShapeDtypeStruct((B, 2 * DIM), jnp.float32),
        ],
        scratch_types=[
            pltpu.VMEM((CH,), jnp.int32),
            pltpu.VMEM((CH, TEXT_DIM), jnp.float32),
            pltpu.VMEM((BPW,), jnp.int32),
            pltpu.VMEM((BPW, 2 * DIM), jnp.float32),
            pltpu.SemaphoreType.DMA,
        ],
    )
    def k(q_hbm, prompt_hbm, ptab_hbm, model_hbm, ctab_hbm, cat_hbm,
          qout_hbm, pout_hbm, vout_hbm,
          idx_c, rows_v, idx_b, small_v, sem):
        wid = lax.axis_index("s") * NC + lax.axis_index("c")
        base = wid * BPW

        @pl.loop(0, BPW // CH)
        def _(j):
            off = base + j * CH
            pltpu.sync_copy(prompt_hbm.at[pl.ds(off, CH)], idx_c)
            pltpu.async_copy(q_hbm.at[idx_c], rows_v, sem).wait()
            pltpu.sync_copy(rows_v, qout_hbm.at[pl.ds(off, CH)])

        pltpu.sync_copy(model_hbm.at[pl.ds(base, BPW)], idx_b)
        pltpu.async_copy(ptab_hbm.at[idx_b], small_v, sem).wait()
        pltpu.sync_copy(small_v, pout_hbm.at[pl.ds(base, BPW)])

        pltpu.sync_copy(cat_hbm.at[pl.ds(base, BPW)], idx_b)
        pltpu.async_copy(ctab_hbm.at[idx_b], small_v, sem).wait()
        pltpu.sync_copy(small_v, vout_hbm.at[pl.ds(base, BPW)])

    return k(Q, prompt, P, model, cat_emb, category)


def _tc_body(qraw_ref, p_ref, v_ref, wt_ref, bt_ref, wc_ref, bc_ref, out_ref):
    q = jnp.dot(qraw_ref[...], wt_ref[...],
                preferred_element_type=jnp.float32) + bt_ref[...]
    p = p_ref[:, :DIM]
    v = v_ref[:, :DIM]
    h = q * (p + v) + p * v
    out_ref[...] = jnp.dot(h, wc_ref[...],
                           preferred_element_type=jnp.float32) + bc_ref[...]


def kernel(model, prompt, category, P, Q, W_text, b_text, cat_emb, W_cls, b_cls):
    P_pad = jnp.pad(P, ((0, 0), (0, DIM)))
    cat_pad = jnp.pad(cat_emb, ((0, 0), (0, DIM)))
    q_raw, p, v = _sc_gather(Q, prompt, P_pad, model, cat_pad, category)

    TB = 2048
    out = pl.pallas_call(
        _tc_body,
        grid=(B // TB,),
        in_specs=[
            pl.BlockSpec((TB, TEXT_DIM), lambda i: (i, 0)),
            pl.BlockSpec((TB, 2 * DIM), lambda i: (i, 0)),
            pl.BlockSpec((TB, 2 * DIM), lambda i: (i, 0)),
            pl.BlockSpec((TEXT_DIM, DIM), lambda i: (0, 0)),
            pl.BlockSpec((1, DIM), lambda i: (0, 0)),
            pl.BlockSpec((DIM, 2), lambda i: (0, 0)),
            pl.BlockSpec((1, 2), lambda i: (0, 0)),
        ],
        out_specs=pl.BlockSpec((TB, 2), lambda i: (i, 0)),
        out_shape=jax.ShapeDtypeStruct((B, 2), jnp.float32),
    )(q_raw, p, v, W_text, b_text.reshape(1, DIM), W_cls, b_cls.reshape(1, 2))
    return out


# speedup vs baseline: 4.2531x; 1.0231x over previous
import functools

import jax
import jax.numpy as jnp
from jax import lax
from jax.experimental import pallas as pl
from jax.experimental.pallas import tpu as pltpu
from jax.experimental.pallas import tpu_sc as plsc

B = 16384
DIM = 64
TEXT_DIM = 768

NC = 2
NS = 16
NW = NC * NS
BPW = B // NW
CH = 64
PCH = 128


def _sc_gather(Q, prompt, P, model, cat_emb, category):
    mesh = plsc.VectorSubcoreMesh(core_axis_name="c", subcore_axis_name="s")

    @functools.partial(
        pl.kernel,
        mesh=mesh,
        out_type=[
            jax.ShapeDtypeStruct((B, TEXT_DIM), jnp.float32),
            jax.ShapeDtypeStruct((B, 2 * DIM), jnp.float32),
            jax.ShapeDtypeStruct((B, 2 * DIM), jnp.float32),
        ],
        scratch_types=[
            pltpu.VMEM((CH,), jnp.int32),
            pltpu.VMEM((CH,), jnp.int32),
            pltpu.VMEM((CH, TEXT_DIM), jnp.float32),
            pltpu.VMEM((CH, TEXT_DIM), jnp.float32),
            pltpu.VMEM((PCH,), jnp.int32),
            pltpu.VMEM((PCH, 2 * DIM), jnp.float32),
            pltpu.SemaphoreType.DMA,
            pltpu.SemaphoreType.DMA,
            pltpu.SemaphoreType.DMA,
        ],
    )
    def k(q_hbm, prompt_hbm, ptab_hbm, model_hbm, ctab_hbm, cat_hbm,
          qout_hbm, pout_hbm, vout_hbm,
          idx0, idx1, rows0, rows1, pidx, prows, sem0, sem1, psem):
        wid = lax.axis_index("s") * NC + lax.axis_index("c")
        base = wid * BPW
        nchunk = BPW // CH
        idx = (idx0, idx1)
        rows = (rows0, rows1)
        sems = (sem0, sem1)

        pltpu.sync_copy(prompt_hbm.at[pl.ds(base, CH)], idx0)
        handle = pltpu.async_copy(q_hbm.at[idx0], rows0, sem0)
        for j in range(nchunk):
            cur = j % 2
            nxt = (j + 1) % 2
            if j + 1 < nchunk:
                off = base + (j + 1) * CH
                pltpu.sync_copy(prompt_hbm.at[pl.ds(off, CH)], idx[nxt])
                nxt_handle = pltpu.async_copy(q_hbm.at[idx[nxt]], rows[nxt],
                                              sems[nxt])
            handle.wait()
            pltpu.sync_copy(rows[cur], qout_hbm.at[pl.ds(base + j * CH, CH)])
            if j + 1 < nchunk:
                handle = nxt_handle

        for tab_hbm, ii_hbm, out_hbm in ((ptab_hbm, model_hbm, pout_hbm),
                                         (ctab_hbm, cat_hbm, vout_hbm)):
            @pl.loop(0, BPW // PCH)
            def _(j):
                off = base + j * PCH
                pltpu.sync_copy(ii_hbm.at[pl.ds(off, PCH)], pidx)
                pltpu.async_copy(tab_hbm.at[pidx], prows, psem).wait()
                pltpu.sync_copy(prows, out_hbm.at[pl.ds(off, PCH)])

    return k(Q, prompt, P, model, cat_emb, category)


def _tc_body(qraw_ref, p_ref, v_ref, wt_ref, bt_ref, wc_ref, bc_ref, out_ref):
    q = jnp.dot(qraw_ref[...], wt_ref[...],
                preferred_element_type=jnp.float32) + bt_ref[...]
    p = p_ref[:, :DIM]
    v = v_ref[:, :DIM]
    h = q * (p + v) + p * v
    out_ref[...] = jnp.dot(h, wc_ref[...],
                           preferred_element_type=jnp.float32) + bc_ref[...]


def kernel(model, prompt, category, P, Q, W_text, b_text, cat_emb, W_cls, b_cls):
    P_pad = jnp.pad(P, ((0, 0), (0, DIM)))
    cat_pad = jnp.pad(cat_emb, ((0, 0), (0, DIM)))
    q_raw, p, v = _sc_gather(Q, prompt, P_pad, model, cat_pad, category)

    TB = 2048
    out = pl.pallas_call(
        _tc_body,
        grid=(B // TB,),
        in_specs=[
            pl.BlockSpec((TB, TEXT_DIM), lambda i: (i, 0)),
            pl.BlockSpec((TB, 2 * DIM), lambda i: (i, 0)),
            pl.BlockSpec((TB, 2 * DIM), lambda i: (i, 0)),
            pl.BlockSpec((TEXT_DIM, DIM), lambda i: (0, 0)),
            pl.BlockSpec((1, DIM), lambda i: (0, 0)),
            pl.BlockSpec((DIM, 2), lambda i: (0, 0)),
            pl.BlockSpec((1, 2), lambda i: (0, 0)),
        ],
        out_specs=pl.BlockSpec((TB, 2), lambda i: (i, 0)),
        out_shape=jax.ShapeDtypeStruct((B, 2), jnp.float32),
    )(q_raw, p, v, W_text, b_text.reshape(1, DIM), W_cls, b_cls.reshape(1, 2))
    return out
